# baseline (speedup 1.0000x reference)
import functools

import jax
import jax.numpy as jnp
from jax import lax
from jax.experimental import pallas as pl
from jax.experimental.pallas import tpu as pltpu
from jax.experimental.pallas import tpu_sc as plsc

_B = 32
_ROWS = 1176
_LANES = 128
_ROW_BLK = 392


def _sc_gather_body(ts_hbm, sa_hbm, so_hbm, sa_out, so_out,
                    idx_v, sa_v, so_v, sem):
    wid = lax.axis_index("s") * 2 + lax.axis_index("c")

    @pl.when(wid == 0)
    def _():
        pltpu.sync_copy(ts_hbm, idx_v)
        pltpu.async_copy(sa_hbm.at[idx_v], sa_v, sem).wait()
        pltpu.async_copy(so_hbm.at[idx_v], so_v, sem).wait()
        pltpu.sync_copy(sa_v, sa_out)
        pltpu.sync_copy(so_v, so_out)


def _sc_gather(time_steps, sa_table, so_table):
    mesh = plsc.VectorSubcoreMesh(core_axis_name="c", subcore_axis_name="s")
    return pl.kernel(
        _sc_gather_body,
        out_type=(
            jax.ShapeDtypeStruct((_B,), jnp.float32),
            jax.ShapeDtypeStruct((_B,), jnp.float32),
        ),
        mesh=mesh,
        scratch_types=(
            pltpu.VMEM((_B,), jnp.int32),
            pltpu.VMEM((_B,), jnp.float32),
            pltpu.VMEM((_B,), jnp.float32),
            pltpu.SemaphoreType.DMA,
        ),
    )(time_steps, sa_table, so_table)


def _tc_combine_body(sa_ref, so_ref, x0_ref, n_ref, o_ref):
    b = pl.program_id(0)
    sa = sa_ref[b]
    so = so_ref[b]
    o_ref[...] = sa * x0_ref[...] + so * n_ref[...]


def _tc_combine(sa_t, so_t, x0, noise):
    grid = (_B, _ROWS // _ROW_BLK)
    blk = pl.BlockSpec((1, _ROW_BLK, _LANES), lambda b, j: (b, j, 0))
    return pl.pallas_call(
        _tc_combine_body,
        grid=grid,
        in_specs=[
            pl.BlockSpec(memory_space=pltpu.SMEM),
            pl.BlockSpec(memory_space=pltpu.SMEM),
            blk,
            blk,
        ],
        out_specs=blk,
        out_shape=jax.ShapeDtypeStruct((_B, _ROWS, _LANES), jnp.float32),
    )(sa_t, so_t, x0, noise)


@jax.jit
def kernel(x0, noise, time_steps, sqrt_alpha_cumprod, sqrt_one_minus_alpha_cumprod):
    shape = x0.shape
    x0r = x0.reshape(_B, _ROWS, _LANES)
    nr = noise.reshape(_B, _ROWS, _LANES)
    ts = time_steps.astype(jnp.int32)
    sa_t, so_t = _sc_gather(ts, sqrt_alpha_cumprod, sqrt_one_minus_alpha_cumprod)
    out = _tc_combine(sa_t, so_t, x0r, nr)
    return out.reshape(shape)


# speedup vs baseline: 1.0754x; 1.0754x over previous
import functools

import jax
import jax.numpy as jnp
from jax import lax
from jax.experimental import pallas as pl
from jax.experimental.pallas import tpu as pltpu
from jax.experimental.pallas import tpu_sc as plsc

_B = 32
_ROWS = 1176
_LANES = 128
_ROW_BLK = 392


def _sc_gather_body(ts_hbm, sa_hbm, so_hbm, sa_out, so_out,
                    idx_v, sa_v, so_v, sem):
    wid = lax.axis_index("s") * 2 + lax.axis_index("c")

    @pl.when(wid == 0)
    def _():
        pltpu.sync_copy(ts_hbm, idx_v)
        pltpu.async_copy(sa_hbm.at[idx_v], sa_v, sem).wait()
        pltpu.async_copy(so_hbm.at[idx_v], so_v, sem).wait()
        pltpu.sync_copy(sa_v, sa_out)
        pltpu.sync_copy(so_v, so_out)


def _sc_gather(time_steps, sa_table, so_table):
    mesh = plsc.VectorSubcoreMesh(core_axis_name="c", subcore_axis_name="s")
    return pl.kernel(
        _sc_gather_body,
        out_type=(
            jax.ShapeDtypeStruct((_B,), jnp.float32),
            jax.ShapeDtypeStruct((_B,), jnp.float32),
        ),
        mesh=mesh,
        scratch_types=(
            pltpu.VMEM((_B,), jnp.int32),
            pltpu.VMEM((_B,), jnp.float32),
            pltpu.VMEM((_B,), jnp.float32),
            pltpu.SemaphoreType.DMA,
        ),
    )(time_steps, sa_table, so_table)


def _tc_combine_body(sa_ref, so_ref, x0_ref, n_ref, o_ref):
    b = pl.program_id(0)
    sa = sa_ref[b]
    so = so_ref[b]
    o_ref[...] = sa * x0_ref[...] + so * n_ref[...]


def _tc_combine(sa_t, so_t, x0, noise):
    grid = (_B, _ROWS // _ROW_BLK)
    blk = pl.BlockSpec((1, _ROW_BLK, _LANES), lambda b, j: (b, j, 0))
    return pl.pallas_call(
        _tc_combine_body,
        grid=grid,
        in_specs=[
            pl.BlockSpec(memory_space=pltpu.SMEM),
            pl.BlockSpec(memory_space=pltpu.SMEM),
            blk,
            blk,
        ],
        out_specs=blk,
        out_shape=jax.ShapeDtypeStruct((_B, _ROWS, _LANES), jnp.float32),
    )(sa_t, so_t, x0, noise)


@jax.jit
def kernel(x0, noise, time_steps, sqrt_alpha_cumprod, sqrt_one_minus_alpha_cumprod):
    shape = x0.shape
    x0r = x0.reshape(_B, _ROWS, _LANES)
    nr = noise.reshape(_B, _ROWS, _LANES)
    ts = time_steps.astype(jnp.int32)
    sa_t = jnp.take(sqrt_alpha_cumprod, ts)
    so_t = jnp.take(sqrt_one_minus_alpha_cumprod, ts)
    out = _tc_combine(sa_t, so_t, x0r, nr)
    return out.reshape(shape)


# speedup vs baseline: 1.5026x; 1.3973x over previous
import functools

import jax
import jax.numpy as jnp
from jax import lax
from jax.experimental import pallas as pl
from jax.experimental.pallas import tpu as pltpu
from jax.experimental.pallas import tpu_sc as plsc

_B = 32
_ROWS = 1176
_LANES = 128
_ROW_BLK = 392


def _sc_gather_body(ts_hbm, sa_hbm, so_hbm, sa_out, so_out,
                    idx_v, sa_v, so_v, sem):
    wid = lax.axis_index("s") * 2 + lax.axis_index("c")

    @pl.when(wid == 0)
    def _():
        pltpu.sync_copy(ts_hbm, idx_v)
        pltpu.async_copy(sa_hbm.at[idx_v], sa_v, sem).wait()
        pltpu.async_copy(so_hbm.at[idx_v], so_v, sem).wait()
        pltpu.sync_copy(sa_v, sa_out)
        pltpu.sync_copy(so_v, so_out)


def _sc_gather(time_steps, sa_table, so_table):
    mesh = plsc.VectorSubcoreMesh(core_axis_name="c", subcore_axis_name="s")
    return pl.kernel(
        _sc_gather_body,
        out_type=(
            jax.ShapeDtypeStruct((_B,), jnp.float32),
            jax.ShapeDtypeStruct((_B,), jnp.float32),
        ),
        mesh=mesh,
        scratch_types=(
            pltpu.VMEM((_B,), jnp.int32),
            pltpu.VMEM((_B,), jnp.float32),
            pltpu.VMEM((_B,), jnp.float32),
            pltpu.SemaphoreType.DMA,
        ),
    )(time_steps, sa_table, so_table)


_SAMPLES_PER_BLK = 4


def _tc_combine_body(sa_ref, so_ref, x0_ref, n_ref, o_ref):
    g = pl.program_id(0)
    for i in range(_SAMPLES_PER_BLK):
        b = g * _SAMPLES_PER_BLK + i
        o_ref[i] = sa_ref[b] * x0_ref[i] + so_ref[b] * n_ref[i]


def _tc_combine(sa_t, so_t, x0, noise):
    s = _SAMPLES_PER_BLK
    grid = (_B // s,)
    blk = pl.BlockSpec((s, _ROWS, _LANES), lambda g: (g, 0, 0))
    return pl.pallas_call(
        _tc_combine_body,
        grid=grid,
        in_specs=[
            pl.BlockSpec(memory_space=pltpu.SMEM),
            pl.BlockSpec(memory_space=pltpu.SMEM),
            blk,
            blk,
        ],
        out_specs=blk,
        out_shape=jax.ShapeDtypeStruct((_B, _ROWS, _LANES), jnp.float32),
    )(sa_t, so_t, x0, noise)


@jax.jit
def kernel(x0, noise, time_steps, sqrt_alpha_cumprod, sqrt_one_minus_alpha_cumprod):
    shape = x0.shape
    x0r = x0.reshape(_B, _ROWS, _LANES)
    nr = noise.reshape(_B, _ROWS, _LANES)
    ts = time_steps.astype(jnp.int32)
    sa_t = jnp.take(sqrt_alpha_cumprod, ts)
    so_t = jnp.take(sqrt_one_minus_alpha_cumprod, ts)
    out = _tc_combine(sa_t, so_t, x0r, nr)
    return out.reshape(shape)


# speedup vs baseline: 6.2891x; 4.1854x over previous
import functools

import jax
import jax.numpy as jnp
from jax import lax
from jax.experimental import pallas as pl
from jax.experimental.pallas import tpu as pltpu
from jax.experimental.pallas import tpu_sc as plsc

_B = 32
_ROWS = 1176
_LANES = 128
_ROW_BLK = 392


def _sc_gather_body(ts_hbm, sa_hbm, so_hbm, sa_out, so_out,
                    idx_v, sa_v, so_v, sem):
    wid = lax.axis_index("s") * 2 + lax.axis_index("c")

    @pl.when(wid == 0)
    def _():
        pltpu.sync_copy(ts_hbm, idx_v)
        pltpu.async_copy(sa_hbm.at[idx_v], sa_v, sem).wait()
        pltpu.async_copy(so_hbm.at[idx_v], so_v, sem).wait()
        pltpu.sync_copy(sa_v, sa_out)
        pltpu.sync_copy(so_v, so_out)


def _sc_gather(time_steps, sa_table, so_table):
    mesh = plsc.VectorSubcoreMesh(core_axis_name="c", subcore_axis_name="s")
    return pl.kernel(
        _sc_gather_body,
        out_type=(
            jax.ShapeDtypeStruct((_B,), jnp.float32),
            jax.ShapeDtypeStruct((_B,), jnp.float32),
        ),
        mesh=mesh,
        scratch_types=(
            pltpu.VMEM((_B,), jnp.int32),
            pltpu.VMEM((_B,), jnp.float32),
            pltpu.VMEM((_B,), jnp.float32),
            pltpu.SemaphoreType.DMA,
        ),
    )(time_steps, sa_table, so_table)


_SAMPLES_PER_BLK = 4


def _tc_combine_body(sa_ref, so_ref, x0_ref, n_ref, o_ref):
    g = pl.program_id(0)
    for i in range(_SAMPLES_PER_BLK):
        b = g * _SAMPLES_PER_BLK + i
        o_ref[i] = sa_ref[b] * x0_ref[i] + so_ref[b] * n_ref[i]


def _tc_combine(sa_t, so_t, x0, noise):
    s = _SAMPLES_PER_BLK
    grid = (_B // s,)
    c, h, w = x0.shape[1:]
    blk = pl.BlockSpec((s, c, h, w), lambda g: (g, 0, 0, 0))
    return pl.pallas_call(
        _tc_combine_body,
        grid=grid,
        in_specs=[
            pl.BlockSpec(memory_space=pltpu.SMEM),
            pl.BlockSpec(memory_space=pltpu.SMEM),
            blk,
            blk,
        ],
        out_specs=blk,
        out_shape=jax.ShapeDtypeStruct(x0.shape, jnp.float32),
    )(sa_t, so_t, x0, noise)


@jax.jit
def kernel(x0, noise, time_steps, sqrt_alpha_cumprod, sqrt_one_minus_alpha_cumprod):
    ts = time_steps.astype(jnp.int32)
    sa_t = jnp.take(sqrt_alpha_cumprod, ts)
    so_t = jnp.take(sqrt_one_minus_alpha_cumprod, ts)
    return _tc_combine(sa_t, so_t, x0, noise)
